# baseline (speedup 1.0000x reference)
import functools

import jax
import jax.numpy as jnp
from jax import lax
from jax.experimental import pallas as pl
from jax.experimental.pallas import tpu as pltpu
from jax.experimental.pallas import tpu_sc as plsc

NUM_EMB = 1000000
DIM = 64
B = 16384 * 20
NW = 32
B_PER_W = B // NW
CHUNK = 512
N_CHUNKS = B_PER_W // CHUNK


def _make_gather():
    mesh = plsc.VectorSubcoreMesh(core_axis_name="c", subcore_axis_name="s")

    @functools.partial(
        pl.kernel,
        mesh=mesh,
        out_type=jax.ShapeDtypeStruct((B, DIM), jnp.float32),
        compiler_params=pltpu.CompilerParams(use_tc_tiling_on_sc=False),
        scratch_types=[
            pltpu.VMEM((B_PER_W,), jnp.int32),
            pltpu.VMEM((CHUNK, DIM), jnp.float32),
            pltpu.VMEM((CHUNK, DIM), jnp.float32),
            pltpu.SemaphoreType.DMA,
            pltpu.SemaphoreType.DMA,
            pltpu.SemaphoreType.DMA,
            pltpu.SemaphoreType.DMA,
        ],
    )
    def gather_kernel(idx_hbm, table_hbm, out_hbm,
                      idx_v, rows0, rows1, g0, g1, o0, o1):
        wid = lax.axis_index("s") * 2 + lax.axis_index("c")
        base = wid * B_PER_W
        pltpu.sync_copy(idx_hbm.at[pl.ds(base, B_PER_W)], idx_v)

        bufs = (rows0, rows1)
        gsems = (g0, g1)
        osems = (o0, o1)

        def start_gather(i):
            return pltpu.async_copy(
                table_hbm.at[idx_v.at[pl.ds(i * CHUNK, CHUNK)]],
                bufs[i % 2], gsems[i % 2])

        def start_out(i):
            return pltpu.async_copy(
                bufs[i % 2], out_hbm.at[pl.ds(base + i * CHUNK, CHUNK)],
                osems[i % 2])

        gd = [None, None]
        od = [None, None]
        gd[0] = start_gather(0)
        for i in range(N_CHUNKS):
            cur = i % 2
            nxt = 1 - cur
            if i + 1 < N_CHUNKS:
                if od[nxt] is not None:
                    od[nxt].wait()
                    od[nxt] = None
                gd[nxt] = start_gather(i + 1)
            gd[cur].wait()
            od[cur] = start_out(i)
        for d in od:
            if d is not None:
                d.wait()

    return gather_kernel


_gather = _make_gather()


@jax.jit
def kernel(indices, weight):
    idx_flat = indices.reshape(-1).astype(jnp.int32)
    out = _gather(idx_flat, weight)
    return out.reshape(indices.shape + (DIM,))


# speedup vs baseline: 1.5172x; 1.5172x over previous
import functools

import jax
import jax.numpy as jnp
from jax import lax
from jax.experimental import pallas as pl
from jax.experimental.pallas import tpu as pltpu
from jax.experimental.pallas import tpu_sc as plsc

NUM_EMB = 1000000
DIM = 64
B = 16384 * 20
NW = 32
B_PER_W = B // NW
CHUNK = 512
N_CHUNKS = B_PER_W // CHUNK


def _make_gather():
    mesh = plsc.VectorSubcoreMesh(core_axis_name="c", subcore_axis_name="s")

    @functools.partial(
        pl.kernel,
        mesh=mesh,
        out_type=jax.ShapeDtypeStruct((B, DIM), jnp.float32),
        compiler_params=pltpu.CompilerParams(use_tc_tiling_on_sc=False),
        scratch_types=[
            pltpu.VMEM((B_PER_W,), jnp.int32),
            pltpu.VMEM((CHUNK, DIM), jnp.float32),
            pltpu.VMEM((CHUNK, DIM), jnp.float32),
            pltpu.SemaphoreType.DMA,
            pltpu.SemaphoreType.DMA,
            pltpu.SemaphoreType.DMA,
            pltpu.SemaphoreType.DMA,
        ],
    )
    def gather_kernel(idx_hbm, table_hbm, out_hbm,
                      idx_v, rows0, rows1, g0, g1, o0, o1):
        wid = lax.axis_index("s") * 2 + lax.axis_index("c")
        base = wid * B_PER_W
        pltpu.sync_copy(idx_hbm.at[pl.ds(base, B_PER_W)], idx_v)

        bufs = (rows0, rows1)
        gsems = (g0, g1)
        osems = (o0, o1)

        def start_gather(i):
            return pltpu.async_copy(
                table_hbm.at[idx_v.at[pl.ds(i * CHUNK, CHUNK)]],
                bufs[i % 2], gsems[i % 2])

        def start_out(i):
            return pltpu.async_copy(
                bufs[i % 2], out_hbm.at[pl.ds(base + i * CHUNK, CHUNK)],
                osems[i % 2])

        gd = [None, None]
        od = [None, None]
        gd[0] = start_gather(0)
        for i in range(N_CHUNKS):
            cur = i % 2
            nxt = 1 - cur
            if i + 1 < N_CHUNKS:
                if od[nxt] is not None:
                    od[nxt].wait()
                    od[nxt] = None
                gd[nxt] = start_gather(i + 1)
            gd[cur].wait()
            od[cur] = start_out(i)
        for d in od:
            if d is not None:
                d.wait()

    return gather_kernel


_gather = _make_gather()

TBLK = 8192
HALF = TBLK // 2
NB = (NUM_EMB + TBLK - 1) // TBLK
LAST_BASE = (NB - 1) * TBLK
TAIL = NUM_EMB - LAST_BASE
TAILH = TAIL // 2


def _transpose_body(wt_ref, out_ref):
    t = wt_ref[...].T
    i = pl.program_id(0)

    @pl.when(i < NB - 1)
    def _full():
        t2 = jnp.concatenate([t[:HALF], t[HALF:]], axis=1)
        out_ref[...] = t2.reshape(-1)

    @pl.when(i == NB - 1)
    def _tail():
        t2 = jnp.concatenate([t[:TAILH], t[TAILH:TAIL]], axis=1)
        out_ref[pl.ds(0, TAIL * DIM)] = t2.reshape(-1)


def _make_transpose():
    return pl.pallas_call(
        _transpose_body,
        grid=(NB,),
        in_specs=[pl.BlockSpec((DIM, TBLK), lambda i: (0, i))],
        out_specs=pl.BlockSpec((TBLK * DIM,), lambda i: (i,)),
        out_shape=jax.ShapeDtypeStruct((NUM_EMB * DIM,), jnp.float32),
    )


_transpose = _make_transpose()


@jax.jit
def kernel(indices, weight):
    idx_flat = indices.reshape(-1).astype(jnp.int32)
    last = idx_flat >= LAST_BASE
    q = jnp.where(last, idx_flat - LAST_BASE, idx_flat % TBLK)
    h = jnp.where(last, TAILH, HALF)
    base = jnp.where(last, LAST_BASE, idx_flat - idx_flat % TBLK)
    idx_perm = base + (q % h) * 2 + q // h
    w_flat = _transpose(weight.T)
    out = _gather(idx_perm, w_flat.reshape(NUM_EMB, DIM))
    return out.reshape(indices.shape + (DIM,))


# speedup vs baseline: 1.8480x; 1.2181x over previous
import functools

import jax
import jax.numpy as jnp
from jax import lax
from jax.experimental import pallas as pl
from jax.experimental.pallas import tpu as pltpu
from jax.experimental.pallas import tpu_sc as plsc

NUM_EMB = 1000000
DIM = 64
B = 16384 * 20
NW = 32
B_PER_W = B // NW
CHUNK = 512
N_CHUNKS = B_PER_W // CHUNK


def _make_gather():
    mesh = plsc.VectorSubcoreMesh(core_axis_name="c", subcore_axis_name="s")

    @functools.partial(
        pl.kernel,
        mesh=mesh,
        out_type=jax.ShapeDtypeStruct((B, DIM), jnp.float32),
        compiler_params=pltpu.CompilerParams(use_tc_tiling_on_sc=False),
        scratch_types=[
            pltpu.VMEM((B_PER_W,), jnp.int32),
            pltpu.VMEM((CHUNK, DIM), jnp.float32),
            pltpu.VMEM((CHUNK, DIM), jnp.float32),
            pltpu.SemaphoreType.DMA,
            pltpu.SemaphoreType.DMA,
            pltpu.SemaphoreType.DMA,
            pltpu.SemaphoreType.DMA,
        ],
    )
    def gather_kernel(idx_hbm, table_hbm, out_hbm,
                      idx_v, rows0, rows1, g0, g1, o0, o1):
        wid = lax.axis_index("s") * 2 + lax.axis_index("c")
        base = wid * B_PER_W
        pltpu.sync_copy(idx_hbm.at[pl.ds(base, B_PER_W)], idx_v)

        bufs = (rows0, rows1)
        gsems = (g0, g1)
        osems = (o0, o1)

        def start_gather(i):
            return pltpu.async_copy(
                table_hbm.at[idx_v.at[pl.ds(i * CHUNK, CHUNK)]],
                bufs[i % 2], gsems[i % 2])

        def start_out(i):
            return pltpu.async_copy(
                bufs[i % 2], out_hbm.at[pl.ds(base + i * CHUNK, CHUNK)],
                osems[i % 2])

        gd = [None, None]
        od = [None, None]
        gd[0] = start_gather(0)
        for i in range(N_CHUNKS):
            cur = i % 2
            nxt = 1 - cur
            if i + 1 < N_CHUNKS:
                if od[nxt] is not None:
                    od[nxt].wait()
                    od[nxt] = None
                gd[nxt] = start_gather(i + 1)
            gd[cur].wait()
            od[cur] = start_out(i)
        for d in od:
            if d is not None:
                d.wait()

    return gather_kernel


_gather = _make_gather()

TBLK = 8192
HALF = TBLK // 2
NB = (NUM_EMB + TBLK - 1) // TBLK
LAST_BASE = (NB - 1) * TBLK
TAIL = NUM_EMB - LAST_BASE
TAILH = TAIL // 2


def _transpose_body(wt_ref, out_ref):
    t = wt_ref[...].T
    i = pl.program_id(0)

    @pl.when(i < NB - 1)
    def _full():
        t2 = jnp.concatenate([t[:HALF], t[HALF:]], axis=1)
        out_ref[...] = t2.reshape(-1)

    @pl.when(i == NB - 1)
    def _tail():
        t2 = jnp.concatenate([t[:TAILH], t[TAILH:TAIL]], axis=1)
        out_ref[pl.ds(0, TAIL * DIM)] = t2.reshape(-1)


def _make_transpose():
    return pl.pallas_call(
        _transpose_body,
        grid=(NB,),
        in_specs=[pl.BlockSpec((DIM, TBLK), lambda i: (0, i))],
        out_specs=pl.BlockSpec((TBLK * DIM,), lambda i: (i,)),
        out_shape=jax.ShapeDtypeStruct((NUM_EMB * DIM,), jnp.float32),
    )


_transpose = _make_transpose()

NSEQ = 16384
NTOK = 20
OBLK = 512


def _out_transpose_body(flat_ref, out_ref):
    m2 = flat_ref[...].reshape(OBLK * NTOK * DIM // 128, 128)
    m2 = m2 + 0.0
    m3 = m2.reshape(OBLK, NTOK // 2, 128)
    for j in range(NTOK):
        half = (j % 2) * DIM
        out_ref[j] = m3[:, j // 2, half:half + DIM].T


def _make_out_transpose():
    return pl.pallas_call(
        _out_transpose_body,
        grid=(NSEQ // OBLK,),
        in_specs=[pl.BlockSpec((OBLK * NTOK * DIM,), lambda i: (i,))],
        out_specs=pl.BlockSpec((NTOK, DIM, OBLK), lambda i: (0, 0, i)),
        out_shape=jax.ShapeDtypeStruct((NTOK, DIM, NSEQ), jnp.float32),
    )


_out_transpose = _make_out_transpose()


@jax.jit
def kernel(indices, weight):
    idx_flat = indices.reshape(-1).astype(jnp.int32)
    last = idx_flat >= LAST_BASE
    q = jnp.where(last, idx_flat - LAST_BASE, idx_flat % TBLK)
    h = jnp.where(last, TAILH, HALF)
    base = jnp.where(last, LAST_BASE, idx_flat - idx_flat % TBLK)
    idx_perm = base + (q % h) * 2 + q // h
    w_flat = _transpose(weight.T)
    out = _gather(idx_perm, w_flat.reshape(NUM_EMB, DIM))
    out_t = _out_transpose(out.reshape(-1))
    return jnp.transpose(out_t, (2, 0, 1))


# speedup vs baseline: 1.9896x; 1.0766x over previous
import functools

import jax
import jax.numpy as jnp
from jax import lax
from jax.experimental import pallas as pl
from jax.experimental.pallas import tpu as pltpu
from jax.experimental.pallas import tpu_sc as plsc

NUM_EMB = 1000000
DIM = 64
B = 16384 * 20
NW = 32
B_PER_W = B // NW
CHUNK = 512
N_CHUNKS = B_PER_W // CHUNK


def _make_gather():
    mesh = plsc.VectorSubcoreMesh(core_axis_name="c", subcore_axis_name="s")

    @functools.partial(
        pl.kernel,
        mesh=mesh,
        out_type=jax.ShapeDtypeStruct((B, DIM), jnp.float32),
        compiler_params=pltpu.CompilerParams(use_tc_tiling_on_sc=False),
        scratch_types=[
            pltpu.VMEM((B_PER_W,), jnp.int32),
            pltpu.VMEM((CHUNK, DIM), jnp.float32),
            pltpu.VMEM((CHUNK, DIM), jnp.float32),
            pltpu.SemaphoreType.DMA,
            pltpu.SemaphoreType.DMA,
            pltpu.SemaphoreType.DMA,
            pltpu.SemaphoreType.DMA,
        ],
    )
    def gather_kernel(idx_hbm, table_hbm, out_hbm,
                      idx_v, rows0, rows1, g0, g1, o0, o1):
        wid = lax.axis_index("s") * 2 + lax.axis_index("c")
        base = wid * B_PER_W
        pltpu.sync_copy(idx_hbm.at[pl.ds(base, B_PER_W)], idx_v)

        bufs = (rows0, rows1)
        gsems = (g0, g1)
        osems = (o0, o1)

        def start_gather(i):
            return pltpu.async_copy(
                table_hbm.at[idx_v.at[pl.ds(i * CHUNK, CHUNK)]],
                bufs[i % 2], gsems[i % 2])

        def start_out(i):
            return pltpu.async_copy(
                bufs[i % 2], out_hbm.at[pl.ds(base + i * CHUNK, CHUNK)],
                osems[i % 2])

        gd = [None, None]
        od = [None, None]
        gd[0] = start_gather(0)
        for i in range(N_CHUNKS):
            cur = i % 2
            nxt = 1 - cur
            if i + 1 < N_CHUNKS:
                if od[nxt] is not None:
                    od[nxt].wait()
                    od[nxt] = None
                gd[nxt] = start_gather(i + 1)
            gd[cur].wait()
            od[cur] = start_out(i)
        for d in od:
            if d is not None:
                d.wait()

    return gather_kernel


_gather = _make_gather()

TBLK = 16384
HALF = TBLK // 2
NB = (NUM_EMB + TBLK - 1) // TBLK
LAST_BASE = (NB - 1) * TBLK
TAIL = NUM_EMB - LAST_BASE
TAILH = TAIL // 2


def _transpose_body(wt_ref, out_ref):
    t = wt_ref[...].T
    i = pl.program_id(0)

    @pl.when(i < NB - 1)
    def _full():
        t2 = jnp.concatenate([t[:HALF], t[HALF:]], axis=1)
        out_ref[...] = t2.reshape(-1)

    @pl.when(i == NB - 1)
    def _tail():
        t2 = jnp.concatenate([t[:TAILH], t[TAILH:TAIL]], axis=1)
        out_ref[pl.ds(0, TAIL * DIM)] = t2.reshape(-1)


def _make_transpose():
    return pl.pallas_call(
        _transpose_body,
        grid=(NB,),
        in_specs=[pl.BlockSpec((DIM, TBLK), lambda i: (0, i))],
        out_specs=pl.BlockSpec((TBLK * DIM,), lambda i: (i,)),
        out_shape=jax.ShapeDtypeStruct((NUM_EMB * DIM,), jnp.float32),
    )


_transpose = _make_transpose()

NSEQ = 16384
NTOK = 20
OBLK = 512


def _out_transpose_body(flat_ref, out_ref):
    m2 = flat_ref[...].reshape(OBLK * NTOK * DIM // 128, 128)
    m2 = m2 + 0.0
    m3 = m2.reshape(OBLK, NTOK // 2, 128)
    for j in range(NTOK):
        half = (j % 2) * DIM
        out_ref[j] = m3[:, j // 2, half:half + DIM].T


def _make_out_transpose():
    return pl.pallas_call(
        _out_transpose_body,
        grid=(NSEQ // OBLK,),
        in_specs=[pl.BlockSpec((OBLK * NTOK * DIM,), lambda i: (i,))],
        out_specs=pl.BlockSpec((NTOK, DIM, OBLK), lambda i: (0, 0, i)),
        out_shape=jax.ShapeDtypeStruct((NTOK, DIM, NSEQ), jnp.float32),
    )


_out_transpose = _make_out_transpose()


@jax.jit
def kernel(indices, weight):
    idx_flat = indices.reshape(-1).astype(jnp.int32)
    last = idx_flat >= LAST_BASE
    q = jnp.where(last, idx_flat - LAST_BASE, idx_flat % TBLK)
    h = jnp.where(last, TAILH, HALF)
    base = jnp.where(last, LAST_BASE, idx_flat - idx_flat % TBLK)
    idx_perm = base + (q % h) * 2 + q // h
    w_flat = _transpose(weight.T)
    out = _gather(idx_perm, w_flat.reshape(NUM_EMB, DIM))
    out_t = _out_transpose(out.reshape(-1))
    return jnp.transpose(out_t, (2, 0, 1))


# speedup vs baseline: 2.0974x; 1.0542x over previous
import functools

import jax
import jax.numpy as jnp
from jax import lax
from jax.experimental import pallas as pl
from jax.experimental.pallas import tpu as pltpu
from jax.experimental.pallas import tpu_sc as plsc

NUM_EMB = 1000000
DIM = 64
B = 16384 * 20
NW = 32
B_PER_W = B // NW
CHUNK = 512
N_CHUNKS = B_PER_W // CHUNK


def _make_gather():
    mesh = plsc.VectorSubcoreMesh(core_axis_name="c", subcore_axis_name="s")

    @functools.partial(
        pl.kernel,
        mesh=mesh,
        out_type=jax.ShapeDtypeStruct((B, DIM), jnp.float32),
        compiler_params=pltpu.CompilerParams(use_tc_tiling_on_sc=False),
        scratch_types=[
            pltpu.VMEM((B_PER_W,), jnp.int32),
            pltpu.VMEM((CHUNK, DIM), jnp.float32),
            pltpu.VMEM((CHUNK, DIM), jnp.float32),
            pltpu.SemaphoreType.DMA,
            pltpu.SemaphoreType.DMA,
            pltpu.SemaphoreType.DMA,
            pltpu.SemaphoreType.DMA,
        ],
    )
    def gather_kernel(idx_hbm, table_hbm, out_hbm,
                      idx_v, rows0, rows1, g0, g1, o0, o1):
        wid = lax.axis_index("s") * 2 + lax.axis_index("c")
        base = wid * B_PER_W
        pltpu.sync_copy(idx_hbm.at[pl.ds(base, B_PER_W)], idx_v)

        bufs = (rows0, rows1)
        gsems = (g0, g1)
        osems = (o0, o1)

        def start_gather(i):
            return pltpu.async_copy(
                table_hbm.at[idx_v.at[pl.ds(i * CHUNK, CHUNK)]],
                bufs[i % 2], gsems[i % 2])

        def start_out(i):
            return pltpu.async_copy(
                bufs[i % 2], out_hbm.at[pl.ds(base + i * CHUNK, CHUNK)],
                osems[i % 2])

        gd = [None, None]
        od = [None, None]
        gd[0] = start_gather(0)
        for i in range(N_CHUNKS):
            cur = i % 2
            nxt = 1 - cur
            if i + 1 < N_CHUNKS:
                if od[nxt] is not None:
                    od[nxt].wait()
                    od[nxt] = None
                gd[nxt] = start_gather(i + 1)
            gd[cur].wait()
            od[cur] = start_out(i)
        for d in od:
            if d is not None:
                d.wait()

    return gather_kernel


_gather = _make_gather()

TBLK = 16384
HALF = TBLK // 2
NB = (NUM_EMB + TBLK - 1) // TBLK
LAST_BASE = (NB - 1) * TBLK
TAIL = NUM_EMB - LAST_BASE
TAILH = TAIL // 2


def _transpose_body(wt_ref, out_ref):
    x = wt_ref[...]
    i = pl.program_id(0)

    @pl.when(i < NB - 1)
    def _full():
        t2 = jnp.concatenate([x[:, :HALF].T, x[:, HALF:].T], axis=1)
        out_ref[...] = t2.reshape(-1)

    @pl.when(i == NB - 1)
    def _tail():
        t2 = jnp.concatenate(
            [x[:, :TAILH].T, x[:, TAILH:TAIL].T], axis=1)
        out_ref[pl.ds(0, TAIL * DIM)] = t2.reshape(-1)


def _make_transpose():
    return pl.pallas_call(
        _transpose_body,
        grid=(NB,),
        in_specs=[pl.BlockSpec((DIM, TBLK), lambda i: (0, i))],
        out_specs=pl.BlockSpec((TBLK * DIM,), lambda i: (i,)),
        out_shape=jax.ShapeDtypeStruct((NUM_EMB * DIM,), jnp.float32),
    )


_transpose = _make_transpose()

NSEQ = 16384
NTOK = 20
OBLK = 512


def _out_transpose_body(flat_ref, out_ref):
    m2 = flat_ref[...].reshape(OBLK * NTOK * DIM // 128, 128)
    m2 = m2 + 0.0
    m3 = m2.reshape(OBLK, NTOK // 2, 128)
    for c in range(NTOK // 2):
        out_ref[2 * c:2 * c + 2] = m3[:, c, :].T.reshape(2, DIM, OBLK)


def _make_out_transpose():
    return pl.pallas_call(
        _out_transpose_body,
        grid=(NSEQ // OBLK,),
        in_specs=[pl.BlockSpec((OBLK * NTOK * DIM,), lambda i: (i,))],
        out_specs=pl.BlockSpec((NTOK, DIM, OBLK), lambda i: (0, 0, i)),
        out_shape=jax.ShapeDtypeStruct((NTOK, DIM, NSEQ), jnp.float32),
    )


_out_transpose = _make_out_transpose()


@jax.jit
def kernel(indices, weight):
    idx_flat = indices.reshape(-1).astype(jnp.int32)
    last = idx_flat >= LAST_BASE
    q = jnp.where(last, idx_flat - LAST_BASE, idx_flat % TBLK)
    h = jnp.where(last, TAILH, HALF)
    base = jnp.where(last, LAST_BASE, idx_flat - idx_flat % TBLK)
    idx_perm = base + (q % h) * 2 + q // h
    w_flat = _transpose(weight.T)
    out = _gather(idx_perm, w_flat.reshape(NUM_EMB, DIM))
    out_t = _out_transpose(out.reshape(-1))
    return jnp.transpose(out_t, (2, 0, 1))


# speedup vs baseline: 2.2296x; 1.0630x over previous
import functools

import jax
import jax.numpy as jnp
from jax import lax
from jax.experimental import pallas as pl
from jax.experimental.pallas import tpu as pltpu
from jax.experimental.pallas import tpu_sc as plsc

NUM_EMB = 1000000
DIM = 64
B = 16384 * 20
NW = 32
B_PER_W = B // NW
CHUNK = 512
N_CHUNKS = B_PER_W // CHUNK

TBLK = 16384
HALF = TBLK // 2
HSHIFT = 13
NB = (NUM_EMB + TBLK - 1) // TBLK
LAST_BASE = (NB - 1) * TBLK
TAIL = NUM_EMB - LAST_BASE
TAILH = TAIL // 2


def _make_gather():
    mesh = plsc.VectorSubcoreMesh(core_axis_name="c", subcore_axis_name="s")

    @functools.partial(
        pl.kernel,
        mesh=mesh,
        out_type=jax.ShapeDtypeStruct((B, DIM), jnp.float32),
        compiler_params=pltpu.CompilerParams(use_tc_tiling_on_sc=False),
        scratch_types=[
            pltpu.VMEM((B_PER_W,), jnp.int32),
            pltpu.VMEM((B_PER_W,), jnp.int32),
            pltpu.VMEM((CHUNK, DIM), jnp.float32),
            pltpu.VMEM((CHUNK, DIM), jnp.float32),
            pltpu.SemaphoreType.DMA,
            pltpu.SemaphoreType.DMA,
            pltpu.SemaphoreType.DMA,
            pltpu.SemaphoreType.DMA,
        ],
    )
    def gather_kernel(idx_hbm, table_hbm, out_hbm,
                      idx_v, idx2_v, rows0, rows1, g0, g1, o0, o1):
        wid = lax.axis_index("s") * 2 + lax.axis_index("c")
        base = wid * B_PER_W
        pltpu.sync_copy(idx_hbm.at[pl.ds(base, B_PER_W)], idx_v)

        bufs = (rows0, rows1)
        gsems = (g0, g1)
        osems = (o0, o1)

        def remap_chunk(i):
            def body(g, carry):
                off = i * CHUNK + g * 16
                v = idx_v[pl.ds(off, 16)]
                last = v >= LAST_BASE
                q = jnp.where(last, v - LAST_BASE, v & (TBLK - 1))
                tail_hi = q >= TAILH
                r = jnp.where(last,
                              jnp.where(tail_hi, q - TAILH, q),
                              q & (HALF - 1))
                hb = jnp.where(last,
                               jnp.where(tail_hi, 1, 0),
                               q >> HSHIFT)
                idx2_v[pl.ds(off, 16)] = (v - q) + r * 2 + hb
                return carry
            lax.fori_loop(0, CHUNK // 16, body, 0)

        def start_gather(i):
            return pltpu.async_copy(
                table_hbm.at[idx2_v.at[pl.ds(i * CHUNK, CHUNK)]],
                bufs[i % 2], gsems[i % 2])

        def start_out(i):
            return pltpu.async_copy(
                bufs[i % 2], out_hbm.at[pl.ds(base + i * CHUNK, CHUNK)],
                osems[i % 2])

        gd = [None, None]
        od = [None, None]
        remap_chunk(0)
        gd[0] = start_gather(0)
        for i in range(N_CHUNKS):
            cur = i % 2
            nxt = 1 - cur
            if i + 1 < N_CHUNKS:
                remap_chunk(i + 1)
                if od[nxt] is not None:
                    od[nxt].wait()
                    od[nxt] = None
                gd[nxt] = start_gather(i + 1)
            gd[cur].wait()
            od[cur] = start_out(i)
        for d in od:
            if d is not None:
                d.wait()

    return gather_kernel


_gather = _make_gather()

def _transpose_body(wt_ref, out_ref):
    x = wt_ref[...]
    i = pl.program_id(0)

    @pl.when(i < NB - 1)
    def _full():
        t2 = jnp.concatenate([x[:, :HALF].T, x[:, HALF:].T], axis=1)
        out_ref[...] = t2.reshape(-1)

    @pl.when(i == NB - 1)
    def _tail():
        t2 = jnp.concatenate(
            [x[:, :TAILH].T, x[:, TAILH:TAIL].T], axis=1)
        out_ref[pl.ds(0, TAIL * DIM)] = t2.reshape(-1)


def _make_transpose():
    return pl.pallas_call(
        _transpose_body,
        grid=(NB,),
        in_specs=[pl.BlockSpec((DIM, TBLK), lambda i: (0, i))],
        out_specs=pl.BlockSpec((TBLK * DIM,), lambda i: (i,)),
        out_shape=jax.ShapeDtypeStruct((NUM_EMB * DIM,), jnp.float32),
    )


_transpose = _make_transpose()

NSEQ = 16384
NTOK = 20
OBLK = 512


def _out_transpose_body(flat_ref, out_ref):
    m2 = flat_ref[...].reshape(OBLK * NTOK * DIM // 128, 128)
    m2 = m2 + 0.0
    m3 = m2.reshape(OBLK, NTOK // 2, 128)
    for c in range(NTOK // 2):
        out_ref[2 * c:2 * c + 2] = m3[:, c, :].T.reshape(2, DIM, OBLK)


def _make_out_transpose():
    return pl.pallas_call(
        _out_transpose_body,
        grid=(NSEQ // OBLK,),
        in_specs=[pl.BlockSpec((OBLK * NTOK * DIM,), lambda i: (i,))],
        out_specs=pl.BlockSpec((NTOK, DIM, OBLK), lambda i: (0, 0, i)),
        out_shape=jax.ShapeDtypeStruct((NTOK, DIM, NSEQ), jnp.float32),
    )


_out_transpose = _make_out_transpose()


@jax.jit
def kernel(indices, weight):
    idx_flat = indices.reshape(-1).astype(jnp.int32)
    w_flat = _transpose(weight.T)
    out = _gather(idx_flat, w_flat.reshape(NUM_EMB, DIM))
    out_t = _out_transpose(out.reshape(-1))
    return jnp.transpose(out_t, (2, 0, 1))


# speedup vs baseline: 2.2789x; 1.0221x over previous
import functools

import jax
import jax.numpy as jnp
from jax import lax
from jax.experimental import pallas as pl
from jax.experimental.pallas import tpu as pltpu
from jax.experimental.pallas import tpu_sc as plsc

NUM_EMB = 1000000
DIM = 64
B = 16384 * 20
NW = 32
B_PER_W = B // NW
CHUNK = 512
N_CHUNKS = B_PER_W // CHUNK

TBLK = 32768
HALF = TBLK // 2
HSHIFT = 14
NB = (NUM_EMB + TBLK - 1) // TBLK
LAST_BASE = (NB - 1) * TBLK
TAIL = NUM_EMB - LAST_BASE
TAILH = TAIL // 2


def _make_gather():
    mesh = plsc.VectorSubcoreMesh(core_axis_name="c", subcore_axis_name="s")

    @functools.partial(
        pl.kernel,
        mesh=mesh,
        out_type=jax.ShapeDtypeStruct((B, DIM), jnp.float32),
        compiler_params=pltpu.CompilerParams(use_tc_tiling_on_sc=False),
        scratch_types=[
            pltpu.VMEM((B_PER_W,), jnp.int32),
            pltpu.VMEM((B_PER_W,), jnp.int32),
            pltpu.VMEM((CHUNK, DIM), jnp.float32),
            pltpu.VMEM((CHUNK, DIM), jnp.float32),
            pltpu.SemaphoreType.DMA,
            pltpu.SemaphoreType.DMA,
            pltpu.SemaphoreType.DMA,
            pltpu.SemaphoreType.DMA,
        ],
    )
    def gather_kernel(idx_hbm, table_hbm, out_hbm,
                      idx_v, idx2_v, rows0, rows1, g0, g1, o0, o1):
        wid = lax.axis_index("s") * 2 + lax.axis_index("c")
        base = wid * B_PER_W
        pltpu.sync_copy(idx_hbm.at[pl.ds(base, B_PER_W)], idx_v)

        bufs = (rows0, rows1)
        gsems = (g0, g1)
        osems = (o0, o1)

        def remap_chunk(i):
            def body(g, carry):
                off = i * CHUNK + g * 16
                v = idx_v[pl.ds(off, 16)]
                last = v >= LAST_BASE
                q = jnp.where(last, v - LAST_BASE, v & (TBLK - 1))
                tail_hi = q >= TAILH
                r = jnp.where(last,
                              jnp.where(tail_hi, q - TAILH, q),
                              q & (HALF - 1))
                hb = jnp.where(last,
                               jnp.where(tail_hi, 1, 0),
                               q >> HSHIFT)
                idx2_v[pl.ds(off, 16)] = (v - q) + r * 2 + hb
                return carry
            lax.fori_loop(0, CHUNK // 16, body, 0)

        def start_gather(i):
            return pltpu.async_copy(
                table_hbm.at[idx2_v.at[pl.ds(i * CHUNK, CHUNK)]],
                bufs[i % 2], gsems[i % 2])

        def start_out(i):
            return pltpu.async_copy(
                bufs[i % 2], out_hbm.at[pl.ds(base + i * CHUNK, CHUNK)],
                osems[i % 2])

        gd = [None, None]
        od = [None, None]
        remap_chunk(0)
        gd[0] = start_gather(0)
        for i in range(N_CHUNKS):
            cur = i % 2
            nxt = 1 - cur
            if i + 1 < N_CHUNKS:
                remap_chunk(i + 1)
                if od[nxt] is not None:
                    od[nxt].wait()
                    od[nxt] = None
                gd[nxt] = start_gather(i + 1)
            gd[cur].wait()
            od[cur] = start_out(i)
        for d in od:
            if d is not None:
                d.wait()

    return gather_kernel


_gather = _make_gather()

def _transpose_body(wt_ref, out_ref):
    x = wt_ref[...]
    i = pl.program_id(0)

    @pl.when(i < NB - 1)
    def _full():
        t2 = jnp.concatenate([x[:, :HALF].T, x[:, HALF:].T], axis=1)
        out_ref[...] = t2.reshape(-1)

    @pl.when(i == NB - 1)
    def _tail():
        t2 = jnp.concatenate(
            [x[:, :TAILH].T, x[:, TAILH:TAIL].T], axis=1)
        out_ref[pl.ds(0, TAIL * DIM)] = t2.reshape(-1)


def _make_transpose():
    return pl.pallas_call(
        _transpose_body,
        grid=(NB,),
        in_specs=[pl.BlockSpec((DIM, TBLK), lambda i: (0, i))],
        out_specs=pl.BlockSpec((TBLK * DIM,), lambda i: (i,)),
        out_shape=jax.ShapeDtypeStruct((NUM_EMB * DIM,), jnp.float32),
        compiler_params=pltpu.CompilerParams(
            vmem_limit_bytes=128 * 1024 * 1024),
    )


_transpose = _make_transpose()

NSEQ = 16384
NTOK = 20
OBLK = 512


def _out_transpose_body(flat_ref, out_ref):
    m2 = flat_ref[...].reshape(OBLK * NTOK * DIM // 128, 128)
    m2 = m2 + 0.0
    m3 = m2.reshape(OBLK, NTOK // 2, 128)
    for c in range(NTOK // 2):
        out_ref[2 * c:2 * c + 2] = m3[:, c, :].T.reshape(2, DIM, OBLK)


def _make_out_transpose():
    return pl.pallas_call(
        _out_transpose_body,
        grid=(NSEQ // OBLK,),
        in_specs=[pl.BlockSpec((OBLK * NTOK * DIM,), lambda i: (i,))],
        out_specs=pl.BlockSpec((NTOK, DIM, OBLK), lambda i: (0, 0, i)),
        out_shape=jax.ShapeDtypeStruct((NTOK, DIM, NSEQ), jnp.float32),
    )


_out_transpose = _make_out_transpose()


@jax.jit
def kernel(indices, weight):
    idx_flat = indices.reshape(-1).astype(jnp.int32)
    w_flat = _transpose(weight.T)
    out = _gather(idx_flat, w_flat.reshape(NUM_EMB, DIM))
    out_t = _out_transpose(out.reshape(-1))
    return jnp.transpose(out_t, (2, 0, 1))


# speedup vs baseline: 2.3429x; 1.0281x over previous
import functools

import jax
import jax.numpy as jnp
from jax import lax
from jax.experimental import pallas as pl
from jax.experimental.pallas import tpu as pltpu
from jax.experimental.pallas import tpu_sc as plsc

NUM_EMB = 1000000
DIM = 64
B = 16384 * 20
NW = 32
B_PER_W = B // NW
CHUNK = 512
N_CHUNKS = B_PER_W // CHUNK

TBLK = 32768
HALF = TBLK // 2
HSHIFT = 14
NB = (NUM_EMB + TBLK - 1) // TBLK
LAST_BASE = (NB - 1) * TBLK
TAIL = NUM_EMB - LAST_BASE
TAILH = TAIL // 2


def _make_gather(bsub):
    b_per_w = bsub // NW
    n_chunks = b_per_w // CHUNK
    mesh = plsc.VectorSubcoreMesh(core_axis_name="c", subcore_axis_name="s")

    @functools.partial(
        pl.kernel,
        mesh=mesh,
        out_type=jax.ShapeDtypeStruct((bsub, DIM), jnp.float32),
        compiler_params=pltpu.CompilerParams(use_tc_tiling_on_sc=False),
        scratch_types=[
            pltpu.VMEM((b_per_w,), jnp.int32),
            pltpu.VMEM((b_per_w,), jnp.int32),
            pltpu.VMEM((CHUNK, DIM), jnp.float32),
            pltpu.VMEM((CHUNK, DIM), jnp.float32),
            pltpu.SemaphoreType.DMA,
            pltpu.SemaphoreType.DMA,
            pltpu.SemaphoreType.DMA,
            pltpu.SemaphoreType.DMA,
        ],
    )
    def gather_kernel(idx_hbm, table_hbm, out_hbm,
                      idx_v, idx2_v, rows0, rows1, g0, g1, o0, o1):
        wid = lax.axis_index("s") * 2 + lax.axis_index("c")
        base = wid * b_per_w
        pltpu.sync_copy(idx_hbm.at[pl.ds(base, b_per_w)], idx_v)

        bufs = (rows0, rows1)
        gsems = (g0, g1)
        osems = (o0, o1)

        def remap_chunk(i):
            def body(g, carry):
                off = i * CHUNK + g * 16
                v = idx_v[pl.ds(off, 16)]
                last = v >= LAST_BASE
                q = jnp.where(last, v - LAST_BASE, v & (TBLK - 1))
                tail_hi = q >= TAILH
                r = jnp.where(last,
                              jnp.where(tail_hi, q - TAILH, q),
                              q & (HALF - 1))
                hb = jnp.where(last,
                               jnp.where(tail_hi, 1, 0),
                               q >> HSHIFT)
                idx2_v[pl.ds(off, 16)] = (v - q) + r * 2 + hb
                return carry
            lax.fori_loop(0, CHUNK // 16, body, 0)

        def start_gather(i):
            return pltpu.async_copy(
                table_hbm.at[idx2_v.at[pl.ds(i * CHUNK, CHUNK)]],
                bufs[i % 2], gsems[i % 2])

        def start_out(i):
            return pltpu.async_copy(
                bufs[i % 2], out_hbm.at[pl.ds(base + i * CHUNK, CHUNK)],
                osems[i % 2])

        gd = [None, None]
        od = [None, None]
        remap_chunk(0)
        gd[0] = start_gather(0)
        for i in range(n_chunks):
            cur = i % 2
            nxt = 1 - cur
            if i + 1 < n_chunks:
                remap_chunk(i + 1)
                if od[nxt] is not None:
                    od[nxt].wait()
                    od[nxt] = None
                gd[nxt] = start_gather(i + 1)
            gd[cur].wait()
            od[cur] = start_out(i)
        for d in od:
            if d is not None:
                d.wait()

    return gather_kernel


_gather_half = _make_gather(B // 2)

def _transpose_body(wt_ref, out_ref):
    x = wt_ref[...]
    i = pl.program_id(0)

    @pl.when(i < NB - 1)
    def _full():
        t2 = jnp.concatenate([x[:, :HALF].T, x[:, HALF:].T], axis=1)
        out_ref[...] = t2.reshape(-1)

    @pl.when(i == NB - 1)
    def _tail():
        t2 = jnp.concatenate(
            [x[:, :TAILH].T, x[:, TAILH:TAIL].T], axis=1)
        out_ref[pl.ds(0, TAIL * DIM)] = t2.reshape(-1)


def _make_transpose():
    return pl.pallas_call(
        _transpose_body,
        grid=(NB,),
        in_specs=[pl.BlockSpec((DIM, TBLK), lambda i: (0, i))],
        out_specs=pl.BlockSpec((TBLK * DIM,), lambda i: (i,)),
        out_shape=jax.ShapeDtypeStruct((NUM_EMB * DIM,), jnp.float32),
        compiler_params=pltpu.CompilerParams(
            vmem_limit_bytes=128 * 1024 * 1024),
    )


_transpose = _make_transpose()

NSEQ = 16384
NTOK = 20
OBLK = 512


def _out_transpose_body(flat_ref, out_ref):
    m2 = flat_ref[...].reshape(OBLK * NTOK * DIM // 128, 128)
    m2 = m2 + 0.0
    m3 = m2.reshape(OBLK, NTOK // 2, 128)
    for c in range(NTOK // 2):
        out_ref[2 * c:2 * c + 2] = m3[:, c, :].T.reshape(2, DIM, OBLK)


NGRID_H = NSEQ // (2 * OBLK)


def _out_transpose_b_body(flat_ref, prev_ref, out_ref):
    del prev_ref
    _out_transpose_body(flat_ref, out_ref)


def _make_out_transpose_a():
    return pl.pallas_call(
        _out_transpose_body,
        grid=(NGRID_H,),
        in_specs=[pl.BlockSpec((OBLK * NTOK * DIM,), lambda i: (i,))],
        out_specs=pl.BlockSpec((NTOK, DIM, OBLK), lambda i: (0, 0, i)),
        out_shape=jax.ShapeDtypeStruct((NTOK, DIM, NSEQ), jnp.float32),
    )


def _make_out_transpose_b():
    return pl.pallas_call(
        _out_transpose_b_body,
        grid=(NGRID_H,),
        in_specs=[
            pl.BlockSpec((OBLK * NTOK * DIM,), lambda i: (i,)),
            pl.BlockSpec(memory_space=pl.ANY),
        ],
        out_specs=pl.BlockSpec((NTOK, DIM, OBLK),
                               lambda i: (0, 0, i + NGRID_H)),
        out_shape=jax.ShapeDtypeStruct((NTOK, DIM, NSEQ), jnp.float32),
        input_output_aliases={1: 0},
    )


_out_transpose_a = _make_out_transpose_a()
_out_transpose_b = _make_out_transpose_b()


@jax.jit
def kernel(indices, weight):
    idx_flat = indices.reshape(-1).astype(jnp.int32)
    w_flat = _transpose(weight.T)
    table = w_flat.reshape(NUM_EMB, DIM)
    out1 = _gather_half(idx_flat[:B // 2], table)
    out2 = _gather_half(idx_flat[B // 2:], table)
    half = _out_transpose_a(out1.reshape(-1))
    full = _out_transpose_b(out2.reshape(-1), half)
    return jnp.transpose(full, (2, 0, 1))


# speedup vs baseline: 2.3675x; 1.0105x over previous
import functools

import jax
import jax.numpy as jnp
from jax import lax
from jax.experimental import pallas as pl
from jax.experimental.pallas import tpu as pltpu
from jax.experimental.pallas import tpu_sc as plsc

NUM_EMB = 1000000
DIM = 64
B = 16384 * 20
NW = 32
B_PER_W = B // NW
CHUNK = 512
N_CHUNKS = B_PER_W // CHUNK

TBLK = 32768
HALF = TBLK // 2
HSHIFT = 14
NB = (NUM_EMB + TBLK - 1) // TBLK
LAST_BASE = (NB - 1) * TBLK
TAIL = NUM_EMB - LAST_BASE
TAILH = TAIL // 2


def _make_gather(bsub):
    b_per_w = bsub // NW
    n_chunks = b_per_w // CHUNK
    mesh = plsc.VectorSubcoreMesh(core_axis_name="c", subcore_axis_name="s")

    @functools.partial(
        pl.kernel,
        mesh=mesh,
        out_type=jax.ShapeDtypeStruct((bsub, DIM), jnp.float32),
        compiler_params=pltpu.CompilerParams(use_tc_tiling_on_sc=False),
        scratch_types=[
            pltpu.VMEM((b_per_w,), jnp.int32),
            pltpu.VMEM((b_per_w,), jnp.int32),
            pltpu.VMEM((CHUNK, DIM), jnp.float32),
            pltpu.VMEM((CHUNK, DIM), jnp.float32),
            pltpu.SemaphoreType.DMA,
            pltpu.SemaphoreType.DMA,
            pltpu.SemaphoreType.DMA,
            pltpu.SemaphoreType.DMA,
        ],
    )
    def gather_kernel(idx_hbm, table_hbm, out_hbm,
                      idx_v, idx2_v, rows0, rows1, g0, g1, o0, o1):
        wid = lax.axis_index("s") * 2 + lax.axis_index("c")
        base = wid * b_per_w
        pltpu.sync_copy(idx_hbm.at[pl.ds(base, b_per_w)], idx_v)

        bufs = (rows0, rows1)
        gsems = (g0, g1)
        osems = (o0, o1)

        def remap_chunk(i):
            def body(g, carry):
                off = i * CHUNK + g * 16
                v = idx_v[pl.ds(off, 16)]
                last = v >= LAST_BASE
                q = jnp.where(last, v - LAST_BASE, v & (TBLK - 1))
                tail_hi = q >= TAILH
                r = jnp.where(last,
                              jnp.where(tail_hi, q - TAILH, q),
                              q & (HALF - 1))
                hb = jnp.where(last,
                               jnp.where(tail_hi, 1, 0),
                               q >> HSHIFT)
                idx2_v[pl.ds(off, 16)] = (v - q) + r * 2 + hb
                return carry
            lax.fori_loop(0, CHUNK // 16, body, 0)

        def start_gather(i):
            return pltpu.async_copy(
                table_hbm.at[idx2_v.at[pl.ds(i * CHUNK, CHUNK)]],
                bufs[i % 2], gsems[i % 2])

        def start_out(i):
            return pltpu.async_copy(
                bufs[i % 2], out_hbm.at[pl.ds(base + i * CHUNK, CHUNK)],
                osems[i % 2])

        gd = [None, None]
        od = [None, None]
        remap_chunk(0)
        gd[0] = start_gather(0)
        for i in range(n_chunks):
            cur = i % 2
            nxt = 1 - cur
            if i + 1 < n_chunks:
                remap_chunk(i + 1)
                if od[nxt] is not None:
                    od[nxt].wait()
                    od[nxt] = None
                gd[nxt] = start_gather(i + 1)
            gd[cur].wait()
            od[cur] = start_out(i)
        for d in od:
            if d is not None:
                d.wait()

    return gather_kernel


_gather_half = _make_gather(B // 2)

def _transpose_body(wt_ref, out_ref):
    x = wt_ref[...]
    i = pl.program_id(0)

    @pl.when(i < NB - 1)
    def _full():
        t2 = jnp.concatenate([x[:, :HALF].T, x[:, HALF:].T], axis=1)
        out_ref[...] = t2.reshape(-1)

    @pl.when(i == NB - 1)
    def _tail():
        t2 = jnp.concatenate(
            [x[:, :TAILH].T, x[:, TAILH:TAIL].T], axis=1)
        out_ref[pl.ds(0, TAIL * DIM)] = t2.reshape(-1)


def _make_transpose():
    return pl.pallas_call(
        _transpose_body,
        grid=(NB,),
        in_specs=[pl.BlockSpec((DIM, TBLK), lambda i: (0, i))],
        out_specs=pl.BlockSpec((TBLK * DIM,), lambda i: (i,)),
        out_shape=jax.ShapeDtypeStruct((NUM_EMB * DIM,), jnp.float32),
        compiler_params=pltpu.CompilerParams(
            vmem_limit_bytes=128 * 1024 * 1024),
    )


_transpose = _make_transpose()

NSEQ = 16384
NTOK = 20
OBLK = 1024


def _out_transpose_body(flat_ref, out_ref):
    m2 = flat_ref[...].reshape(OBLK * NTOK * DIM // 128, 128)
    m2 = m2 + 0.0
    m3 = m2.reshape(OBLK, NTOK // 2, 128)
    for c in range(NTOK // 2):
        out_ref[2 * c:2 * c + 2] = m3[:, c, :].T.reshape(2, DIM, OBLK)


NGRID_H = NSEQ // (2 * OBLK)


def _out_transpose_b_body(flat_ref, prev_ref, out_ref):
    del prev_ref
    _out_transpose_body(flat_ref, out_ref)


def _make_out_transpose_a():
    return pl.pallas_call(
        _out_transpose_body,
        grid=(NGRID_H,),
        in_specs=[pl.BlockSpec((OBLK * NTOK * DIM,), lambda i: (i,))],
        out_specs=pl.BlockSpec((NTOK, DIM, OBLK), lambda i: (0, 0, i)),
        out_shape=jax.ShapeDtypeStruct((NTOK, DIM, NSEQ), jnp.float32),
        compiler_params=pltpu.CompilerParams(
            vmem_limit_bytes=60 * 1024 * 1024),
    )


def _make_out_transpose_b():
    return pl.pallas_call(
        _out_transpose_b_body,
        grid=(NGRID_H,),
        in_specs=[
            pl.BlockSpec((OBLK * NTOK * DIM,), lambda i: (i,)),
            pl.BlockSpec(memory_space=pl.ANY),
        ],
        out_specs=pl.BlockSpec((NTOK, DIM, OBLK),
                               lambda i: (0, 0, i + NGRID_H)),
        out_shape=jax.ShapeDtypeStruct((NTOK, DIM, NSEQ), jnp.float32),
        input_output_aliases={1: 0},
        compiler_params=pltpu.CompilerParams(
            vmem_limit_bytes=60 * 1024 * 1024),
    )


_out_transpose_a = _make_out_transpose_a()
_out_transpose_b = _make_out_transpose_b()


@jax.jit
def kernel(indices, weight):
    idx_flat = indices.reshape(-1).astype(jnp.int32)
    w_flat = _transpose(weight.T)
    table = w_flat.reshape(NUM_EMB, DIM)
    out1 = _gather_half(idx_flat[:B // 2], table)
    out2 = _gather_half(idx_flat[B // 2:], table)
    half = _out_transpose_a(out1.reshape(-1))
    full = _out_transpose_b(out2.reshape(-1), half)
    return jnp.transpose(full, (2, 0, 1))
